# Initial kernel scaffold; baseline (speedup 1.0000x reference)
import jax
import jax.numpy as jnp
from jax.experimental import pallas as pl


def kernel(x_vul, x_wp, x_nn, W_p2v, W_n2v, W_v2w, W_v2n, Wn_vul, bn_vul, Wn_wp, bn_wp, Wn_nn, bn_nn, edge_index_p, edge_index_n, edge_index_vw, edge_index_vn):
    raise NotImplementedError("write your pallas kernel here")


# speedup vs baseline: 1.3103x; 1.3103x over previous
import functools

import jax
import jax.numpy as jnp
from jax import lax
from jax.experimental import pallas as pl
from jax.experimental.pallas import tpu as pltpu

N = 50000
E = 400000
D = 128
RB = 2000


def _tc_proj_body(xv, xw, xn, wp2v, wn2v, wv2w, wv2n, wnv, bnv, wnw, bnw, wnn, bnn,
                  o_htv, o_htw, o_htn, o_hrwp, o_hrnn, o_hrvw, o_hrvf):
    def proj(x, w, b=None):
        h = jnp.dot(x, w.T, preferred_element_type=jnp.float32,
                    precision=lax.Precision.HIGHEST)
        if b is not None:
            h = h + b
        return h

    o_htv[...] = proj(xv[...], wnv[...], bnv[...])
    o_htw[...] = proj(xw[...], wnw[...], bnw[...])
    o_htn[...] = proj(xn[...], wnn[...], bnn[...])
    o_hrwp[...] = proj(xw[...], wp2v[...])
    o_hrnn[...] = proj(xn[...], wn2v[...])
    o_hrvw[...] = proj(xv[...], wv2w[...])
    o_hrvf[...] = proj(xv[...], wv2n[...])


def _tc_projections(x_vul, x_wp, x_nn, W_p2v, W_n2v, W_v2w, W_v2n,
                    Wn_vul, bn_vul, Wn_wp, bn_wp, Wn_nn, bn_nn):
    row_spec = pl.BlockSpec((RB, D), lambda i: (i, 0))
    w_spec = pl.BlockSpec((D, D), lambda i: (0, 0))
    b_spec = pl.BlockSpec((1, D), lambda i: (0, 0))
    out_sd = jax.ShapeDtypeStruct((N, D), jnp.float32)
    return pl.pallas_call(
        _tc_proj_body,
        grid=(N // RB,),
        in_specs=[row_spec, row_spec, row_spec,
                  w_spec, w_spec, w_spec, w_spec,
                  w_spec, b_spec, w_spec, b_spec, w_spec, b_spec],
        out_specs=[row_spec] * 7,
        out_shape=[out_sd] * 7,
    )(x_vul, x_wp, x_nn, W_p2v, W_n2v, W_v2w, W_v2n,
      Wn_vul, bn_vul.reshape(1, D), Wn_wp, bn_wp.reshape(1, D),
      Wn_nn, bn_nn.reshape(1, D))


def _cos(a, b):
    dot = jnp.sum(a * b, axis=1)
    na = jnp.sqrt(jnp.sum(a * a, axis=1))
    nb = jnp.sqrt(jnp.sum(b * b, axis=1))
    return (dot / jnp.maximum(na * nb, 1e-8))[:, None]


def kernel(x_vul, x_wp, x_nn, W_p2v, W_n2v, W_v2w, W_v2n,
           Wn_vul, bn_vul, Wn_wp, bn_wp, Wn_nn, bn_nn,
           edge_index_p, edge_index_n, edge_index_vw, edge_index_vn):
    ht_vul, ht_wp, ht_nn, hr_wp, hr_nn, hr_vul_w, hr_vul_final = _tc_projections(
        x_vul, x_wp, x_nn, W_p2v, W_n2v, W_v2w, W_v2n,
        Wn_vul, bn_vul, Wn_wp, bn_wp, Wn_nn, bn_nn)

    sp, dp = edge_index_p[0], edge_index_p[1]
    sn, dn = edge_index_n[0], edge_index_n[1]
    sw, dw = edge_index_vw[0], edge_index_vw[1]
    sv, dv = edge_index_vn[0], edge_index_vn[1]

    seg = lambda data, s: jax.ops.segment_sum(data, s, num_segments=N)

    s_p = _cos(hr_wp[sp], ht_vul[dp])
    s_n = _cos(hr_nn[sn], ht_vul[dn])
    s_n = jnp.where(s_n > 0.7, s_n * 0.5, s_n) * 0.2
    s_vw = _cos(hr_vul_w[sw], ht_wp[dw])
    s_vn = _cos(hr_vul_final[sv], ht_nn[dv])

    h_vul = 0.6 * seg(s_p * hr_wp[sp], dp) + 0.4 * seg(s_n * hr_nn[sn], dn)
    h_wp = seg(s_vw * hr_vul_final[sw], dw)
    h_nn = seg(s_vn * hr_vul_final[sv], dv)

    out_vul = jnp.concatenate([ht_vul, h_vul], axis=1)
    out_wp = jnp.concatenate([ht_wp, h_wp], axis=1)
    out_nn = jnp.concatenate([ht_nn, h_nn], axis=1)
    return jnp.concatenate([out_vul, out_wp, out_nn], axis=0)
